# speedup vs baseline: 3.6742x; 3.6742x over previous
import functools

import jax
import jax.numpy as jnp
from jax.experimental import pallas as pl
from jax.experimental.pallas import tpu as pltpu

_EPS = 1e-5


def _pow2_tile(m, cap):
    t = 8
    while t < cap and m % (2 * t) == 0:
        t *= 2
    return t


def _gram_kernel(x_ref, g_ref, s_ref):
    x = x_ref[...]
    g_ref[0] = jax.lax.dot_general(
        x, x, (((0,), (0,)), ((), ())), preferred_element_type=jnp.float32)
    s_ref[0] = jnp.sum(x, axis=0, keepdims=True)


def _fused_main_kernel(xs_ref, w1_ref, sh1_ref, w2_ref,
                       y2_ref, s_ref, q_ref, *, wo):
    xb = xs_ref[...].astype(jnp.bfloat16)
    a1 = jnp.dot(xb, w1_ref[...], preferred_element_type=jnp.float32)
    a1 = jnp.maximum(a1 + sh1_ref[...], 0.0).astype(jnp.bfloat16)
    r, f1 = a1.shape
    col = jax.lax.broadcasted_iota(jnp.int32, (r, 1), 0) % wo
    zero = jnp.zeros((), jnp.bfloat16)
    left = jnp.where(col >= 1, jnp.roll(a1, 1, axis=0), zero)
    right = jnp.where(col < wo - 1, jnp.roll(a1, -1, axis=0), zero)
    pmid = jnp.concatenate([left, a1, right], axis=1)
    zrow = jnp.zeros((wo, 3 * f1), jnp.bfloat16)
    p = jnp.concatenate([zrow, pmid, zrow], axis=0)
    acc = jnp.dot(p[0:r], w2_ref[0], preferred_element_type=jnp.float32)
    acc = acc + jnp.dot(p[wo:wo + r], w2_ref[1],
                        preferred_element_type=jnp.float32)
    acc = acc + jnp.dot(p[2 * wo:2 * wo + r], w2_ref[2],
                        preferred_element_type=jnp.float32)
    y2_ref[...] = acc
    s_ref[0] = jnp.sum(acc, axis=0, keepdims=True)
    q_ref[0] = jnp.sum(acc * acc, axis=0, keepdims=True)


def _bn_relu_matmul_stats_kernel(y_ref, sc_ref, sh_ref, w_ref,
                                 o_ref, s_ref, q_ref):
    a = jnp.maximum(y_ref[...] * sc_ref[...] + sh_ref[...], 0.0)
    y = jnp.dot(a.astype(jnp.bfloat16), w_ref[...],
                preferred_element_type=jnp.float32)
    o_ref[...] = y
    s_ref[0] = jnp.sum(y, axis=0, keepdims=True)
    q_ref[0] = jnp.sum(y * y, axis=0, keepdims=True)


def _final_kernel(y3_ref, xs_ref, wsc_ref, sc3_ref, sh3_ref, shc_ref, o_ref):
    sc = jnp.dot(xs_ref[...].astype(jnp.bfloat16), wsc_ref[...],
                 preferred_element_type=jnp.float32)
    o_ref[...] = jnp.maximum(
        y3_ref[...] * sc3_ref[...] + sh3_ref[...] + sc + shc_ref[...], 0.0)


def kernel(x_nchw, w1, b1, w2, b2, w3, b3, wsc, bsc,
           g1, be1, g2, be2, g3, be3, gsc, besc):
    stride = 2
    x = x_nchw[:, :, ::stride, ::stride]
    x = jnp.transpose(x, (0, 2, 3, 1)).astype(jnp.float32)
    N, Ho, Wo, C = x.shape
    F1 = w1.shape[1]
    F2 = w2.shape[3]
    F3 = w3.shape[1]
    M = N * Ho * Wo
    R = Ho * Wo
    xs2d = x.reshape(M, C)

    par = pltpu.CompilerParams(dimension_semantics=("parallel",))

    TG = _pow2_tile(M, 2048)
    ng = M // TG
    gram_p, sum_p = pl.pallas_call(
        _gram_kernel,
        out_shape=(jax.ShapeDtypeStruct((ng, C, C), jnp.float32),
                   jax.ShapeDtypeStruct((ng, 1, C), jnp.float32)),
        grid=(ng,),
        in_specs=[pl.BlockSpec((TG, C), lambda i: (i, 0))],
        out_specs=(pl.BlockSpec((1, C, C), lambda i: (i, 0, 0)),
                   pl.BlockSpec((1, 1, C), lambda i: (i, 0, 0))),
        compiler_params=par,
    )(xs2d)
    gram = jnp.sum(gram_p, axis=0)
    sv = jnp.sum(sum_p, axis=(0, 1))

    def bn_linear(w, g, be):
        mean = (sv @ w) / M
        e2 = jnp.sum((gram @ w) * w, axis=0) / M
        var = jnp.maximum(e2 - mean * mean, 0.0)
        scale = g * jax.lax.rsqrt(var + _EPS)
        shift = be - mean * scale
        return scale, shift

    sc1, sh1 = bn_linear(w1, g1, be1)
    scc, shc = bn_linear(wsc, gsc, besc)
    w1f = (w1 * sc1[None, :]).astype(jnp.bfloat16)
    wscf = (wsc * scc[None, :]).astype(jnp.bfloat16)
    w2r = w2.reshape(w2.shape[0], w2.shape[1] * F1, F2).astype(jnp.bfloat16)

    y2, s2p, q2p = pl.pallas_call(
        functools.partial(_fused_main_kernel, wo=Wo),
        out_shape=(jax.ShapeDtypeStruct((M, F2), jnp.float32),
                   jax.ShapeDtypeStruct((N, 1, F2), jnp.float32),
                   jax.ShapeDtypeStruct((N, 1, F2), jnp.float32)),
        grid=(N,),
        in_specs=[pl.BlockSpec((R, C), lambda i: (i, 0)),
                  pl.BlockSpec((C, F1), lambda i: (0, 0)),
                  pl.BlockSpec((1, F1), lambda i: (0, 0)),
                  pl.BlockSpec((3, 3 * F1, F2), lambda i: (0, 0, 0))],
        out_specs=(pl.BlockSpec((R, F2), lambda i: (i, 0)),
                   pl.BlockSpec((1, 1, F2), lambda i: (i, 0, 0)),
                   pl.BlockSpec((1, 1, F2), lambda i: (i, 0, 0))),
        compiler_params=par,
    )(xs2d, w1f, sh1.reshape(1, -1), w2r)

    s2 = jnp.sum(s2p, axis=(0, 1))
    q2 = jnp.sum(q2p, axis=(0, 1))
    mean2 = s2 / M
    var2 = jnp.maximum(q2 / M - mean2 * mean2, 0.0)
    sc2 = g2 * jax.lax.rsqrt(var2 + _EPS)
    sh2 = be2 - mean2 * sc2

    TK = _pow2_tile(M, 2048)
    nk = M // TK
    y3, s3p, q3p = pl.pallas_call(
        _bn_relu_matmul_stats_kernel,
        out_shape=(jax.ShapeDtypeStruct((M, F3), jnp.float32),
                   jax.ShapeDtypeStruct((nk, 1, F3), jnp.float32),
                   jax.ShapeDtypeStruct((nk, 1, F3), jnp.float32)),
        grid=(nk,),
        in_specs=[pl.BlockSpec((TK, F2), lambda i: (i, 0)),
                  pl.BlockSpec((1, F2), lambda i: (0, 0)),
                  pl.BlockSpec((1, F2), lambda i: (0, 0)),
                  pl.BlockSpec((F2, F3), lambda i: (0, 0))],
        out_specs=(pl.BlockSpec((TK, F3), lambda i: (i, 0)),
                   pl.BlockSpec((1, 1, F3), lambda i: (i, 0, 0)),
                   pl.BlockSpec((1, 1, F3), lambda i: (i, 0, 0))),
        compiler_params=par,
    )(y2, sc2.reshape(1, -1), sh2.reshape(1, -1), w3.astype(jnp.bfloat16))

    s3 = jnp.sum(s3p, axis=(0, 1))
    q3 = jnp.sum(q3p, axis=(0, 1))
    mean3 = s3 / M
    var3 = jnp.maximum(q3 / M - mean3 * mean3, 0.0)
    sc3 = g3 * jax.lax.rsqrt(var3 + _EPS)
    sh3 = be3 - mean3 * sc3

    out2d = pl.pallas_call(
        _final_kernel,
        out_shape=jax.ShapeDtypeStruct((M, F3), jnp.float32),
        grid=(nk,),
        in_specs=[pl.BlockSpec((TK, F3), lambda i: (i, 0)),
                  pl.BlockSpec((TK, C), lambda i: (i, 0)),
                  pl.BlockSpec((C, F3), lambda i: (0, 0)),
                  pl.BlockSpec((1, F3), lambda i: (0, 0)),
                  pl.BlockSpec((1, F3), lambda i: (0, 0)),
                  pl.BlockSpec((1, F3), lambda i: (0, 0))],
        out_specs=pl.BlockSpec((TK, F3), lambda i: (i, 0)),
        compiler_params=par,
    )(y3, xs2d, wscf, sc3.reshape(1, -1), sh3.reshape(1, -1),
      shc.reshape(1, -1))

    out = out2d.reshape(N, Ho, Wo, F3)
    return jnp.transpose(out, (0, 3, 1, 2))
